# Initial kernel scaffold; baseline (speedup 1.0000x reference)
import jax
import jax.numpy as jnp
from jax.experimental import pallas as pl


def kernel(input_patch, pos_table):
    raise NotImplementedError("write your pallas kernel here")


# speedup vs baseline: 1.0482x; 1.0482x over previous
import jax
import jax.numpy as jnp
from jax.experimental import pallas as pl

_BB = 2


def _add_kernel(x_ref, pos_ref, o_ref):
    o_ref[...] = x_ref[...] + pos_ref[...][None, :, :]


def kernel(input_patch, pos_table):
    B, P, D = input_patch.shape
    grid = (B // _BB,)
    return pl.pallas_call(
        _add_kernel,
        grid=grid,
        in_specs=[
            pl.BlockSpec((_BB, P, D), lambda i: (i, 0, 0)),
            pl.BlockSpec((P, D), lambda i: (0, 0)),
        ],
        out_specs=pl.BlockSpec((_BB, P, D), lambda i: (i, 0, 0)),
        out_shape=jax.ShapeDtypeStruct((B, P, D), input_patch.dtype),
    )(input_patch, pos_table)


# speedup vs baseline: 1.0572x; 1.0086x over previous
import jax
import jax.numpy as jnp
from jax.experimental import pallas as pl

_BB = 4


def _add_kernel(x_ref, pos_ref, o_ref):
    o_ref[...] = x_ref[...] + pos_ref[...][None, :, :]


def kernel(input_patch, pos_table):
    B, P, D = input_patch.shape
    grid = (B // _BB,)
    return pl.pallas_call(
        _add_kernel,
        grid=grid,
        in_specs=[
            pl.BlockSpec((_BB, P, D), lambda i: (i, 0, 0)),
            pl.BlockSpec((P, D), lambda i: (0, 0)),
        ],
        out_specs=pl.BlockSpec((_BB, P, D), lambda i: (i, 0, 0)),
        out_shape=jax.ShapeDtypeStruct((B, P, D), input_patch.dtype),
    )(input_patch, pos_table)
